# baseline (speedup 1.0000x reference)
import numpy as np

import jax
import jax.numpy as jnp
from jax import lax
from jax.experimental import pallas as pl
from jax.experimental.pallas import tpu as pltpu
from jax.experimental.pallas import tpu_sc as plsc

ROWS = 128
N = 32768
L = 16
NV = N // L
NC = 2
NS = 16
NW = NC * NS
RPW = ROWS // NW

NB = 2048
SHIFTS = (0, 11, 22)
MASKS = (2047, 2047, 1023)
NBINS = (2048, 2048, 1024)

C = 2
CV = NV // C
CH_SHIFT = 14

UNROLL = 32

MININT = np.int32(-2147483648)


def _to_key(v):
    m = v >> 31
    return v ^ (m | MININT)


def _from_key(k):
    m = k >> 31
    return k ^ (~m | MININT)


def _digit(k, p):
    return lax.shift_right_logical(k, jnp.int32(SHIFTS[p])) & jnp.int32(MASKS[p])


def _zero(ref, n):
    zeros = jnp.zeros((L,), jnp.int32)

    def body(i, c):
        ref[pl.ds(i * L, L)] = zeros
        return c

    lax.fori_loop(0, n // L, body, 0)


def _body(x_hbm, out_hbm, buf_a, buf_b, buf_c, h0, h1, h2, gtmp,
          ptr_lo, ptr_hi, sem_in, sem_out):
    wid = lax.axis_index("s") * NC + lax.axis_index("c")
    bufs = (buf_a, buf_b, buf_c)
    hists = (h0, h1, h2)
    ptrs = (ptr_lo, ptr_hi)
    row0 = wid * RPW
    zeros = jnp.zeros((L,), jnp.int32)

    for h in hists:
        _zero(h, C * NB)

    def build_ptrs(cnt, nbins):
        def totals(i, c):
            ds = pl.ds(i * L, L)
            gtmp[ds] = cnt[pl.ds(i * L, L)] + cnt[pl.ds(NB + i * L, L)]
            return c

        lax.fori_loop(0, nbins // L, totals, 0)

        def excl(i, carry):
            ds = pl.ds(i * L, L)
            h = gtmp[ds]
            inc = plsc.cumsum(h)
            gtmp[ds] = inc - h + carry
            return carry + jnp.max(inc)

        lax.fori_loop(0, nbins // L, excl, jnp.int32(0))

        def bases(i, c):
            ds = pl.ds(i * L, L)
            run = gtmp[ds]
            ptr_lo[ds] = run
            ptr_hi[ds] = run + cnt[pl.ds(i * L, L)]
            cnt[pl.ds(i * L, L)] = zeros
            cnt[pl.ds(NB + i * L, L)] = zeros
            return c

        lax.fori_loop(0, nbins // L, bases, 0)

    def sort_row(src0, pong):
        def sweep0(i, c):
            slots = [(cix, pl.ds((cix * CV + i * (UNROLL // C) + j) * L, L))
                     for j in range(UNROLL // C) for cix in range(C)]
            ks = [_to_key(src0[sl]) for _, sl in slots]
            digs = [_digit(k, 0) for k in ks]
            scans = [plsc.scan_count(d) for d in digs]
            for s, (cix, _) in enumerate(slots):
                cnt, lastm = scans[s]
                plsc.addupdate_scatter(h0, [digs[s] + jnp.int32(cix * NB)],
                                       cnt, mask=lastm)
            return c

        lax.fori_loop(0, CV // (UNROLL // C), sweep0, 0)

        def permute(p, src, dst, hist, hist_next):
            build_ptrs(hist, NBINS[p])

            def sweep(i, c):
                slots = [(cix, cix * CV + i * (UNROLL // C) + j)
                         for j in range(UNROLL // C) for cix in range(C)]
                raw = [src[pl.ds(iv * L, L)] for _, iv in slots]
                ks = [_to_key(v) for v in raw] if p == 0 else raw
                digs = [_digit(k, p) for k in ks]
                scans = [plsc.scan_count(d) for d in digs]
                vals = ks if p < 2 else [_from_key(k) for k in ks]
                offs = []
                for s, (cix, _) in enumerate(slots):
                    cnt, lastm = scans[s]
                    d = digs[s]
                    base = plsc.load_gather(ptrs[cix], [d])
                    nxt = base + cnt
                    plsc.store_scatter(ptrs[cix], [d], nxt, mask=lastm)
                    plsc.store_scatter(dst, [nxt - 1], vals[s])
                    offs.append(nxt - 1)
                if hist_next is not None:
                    idx2s = []
                    for s, k in enumerate(ks):
                        d2 = _digit(k, p + 1)
                        idx2 = lax.shift_left(
                            lax.shift_right_logical(offs[s],
                                                    jnp.int32(CH_SHIFT)),
                            jnp.int32(11)) | d2
                        idx2s.append(idx2)
                    scans2 = [plsc.scan_count(ix) for ix in idx2s]
                    for ix, (cnt2, last2) in zip(idx2s, scans2):
                        plsc.addupdate_scatter(hist_next, [ix], cnt2,
                                               mask=last2)
                return c

            lax.fori_loop(0, CV // (UNROLL // C), sweep, 0)

        permute(0, src0, pong, h0, h1)
        permute(1, pong, src0, h1, h2)
        permute(2, src0, pong, h2, None)

    sched_x = [0, 2, 1, 0]
    sched_y = [1, 0, 2, 1]
    in_h = {0: pltpu.async_copy(x_hbm.at[row0], bufs[0], sem_in)}
    out_h = {}
    for r in range(RPW):
        x_buf = bufs[sched_x[r]]
        y_buf = bufs[sched_y[r]]
        in_h[r].wait()
        if r >= 1:
            out_h[r - 1].wait()
        if r + 1 < RPW:
            in_h[r + 1] = pltpu.async_copy(
                x_hbm.at[row0 + (r + 1)], bufs[sched_x[r + 1]], sem_in)
        sort_row(x_buf, y_buf)
        out_h[r] = pltpu.async_copy(y_buf, out_hbm.at[row0 + r], sem_out)
    out_h[RPW - 1].wait()


@jax.jit
def kernel(x):
    xi = lax.bitcast_convert_type(x, jnp.int32)
    mesh = plsc.VectorSubcoreMesh(core_axis_name="c", subcore_axis_name="s")
    sort_rows = pl.kernel(
        _body,
        out_type=jax.ShapeDtypeStruct((ROWS, N), jnp.int32),
        mesh=mesh,
        compiler_params=pltpu.CompilerParams(needs_layout_passes=False),
        scratch_types=[
            pltpu.VMEM((N,), jnp.int32),
            pltpu.VMEM((N,), jnp.int32),
            pltpu.VMEM((N,), jnp.int32),
            pltpu.VMEM((C * NB,), jnp.int32),
            pltpu.VMEM((C * NB,), jnp.int32),
            pltpu.VMEM((C * NB,), jnp.int32),
            pltpu.VMEM((NB,), jnp.int32),
            pltpu.VMEM((NB,), jnp.int32),
            pltpu.VMEM((NB,), jnp.int32),
            pltpu.SemaphoreType.DMA,
            pltpu.SemaphoreType.DMA,
        ],
    )
    oi = sort_rows(xi)
    return lax.bitcast_convert_type(oi, jnp.float32)


# speedup vs baseline: 1.1036x; 1.1036x over previous
import numpy as np

import jax
import jax.numpy as jnp
from jax import lax
from jax.experimental import pallas as pl
from jax.experimental.pallas import tpu as pltpu
from jax.experimental.pallas import tpu_sc as plsc

ROWS = 128
N = 32768
L = 16
NV = N // L
NC = 2
NS = 16
NW = NC * NS
RPW = ROWS // NW

NB = 2048
SHIFTS = (0, 11, 22)
MASKS = (2047, 2047, 1023)
NBINS = (2048, 2048, 1024)

MININT = np.int32(-2147483648)


def _to_key(v):
    m = v >> 31
    return v ^ (m | MININT)


def _from_key(k):
    m = k >> 31
    return k ^ (~m | MININT)


def _digit(k, p):
    return lax.shift_right_logical(k, jnp.int32(SHIFTS[p])) & jnp.int32(MASKS[p])


def _zero_hist(hist, nbins):
    zeros = jnp.zeros((L,), jnp.int32)

    def body(i, c):
        hist[pl.ds(i * L, L)] = zeros
        return c

    lax.fori_loop(0, nbins // L, body, 0)


def _exclusive_scan(hist, nbins):
    def body(i, carry):
        h = hist[pl.ds(i * L, L)]
        inc = plsc.cumsum(h)
        hist[pl.ds(i * L, L)] = inc - h + carry
        return carry + jnp.sum(h)

    lax.fori_loop(0, nbins // L, body, jnp.int32(0))


UNROLL = 32


def _body(x_hbm, out_hbm, buf_a, buf_b, buf_c, hist_0, hist_1, hist_2,
          sem_in, sem_out):
    wid = lax.axis_index("s") * NC + lax.axis_index("c")
    hists = (hist_0, hist_1, hist_2)
    bufs = (buf_a, buf_b, buf_c)
    row0 = wid * RPW

    def sort_row(src0, pong):
        for p in range(3):
            _zero_hist(hists[p], NBINS[p])

        def sweep0(i, c):
            ks = []
            for u in range(UNROLL):
                v = src0[pl.ds((i * UNROLL + u) * L, L)]
                ks.append(_to_key(v))
            digs = [[_digit(k, p) for k in ks] for p in range(3)]
            for p in range(3):
                scans = [plsc.scan_count(d) for d in digs[p]]
                for u in range(UNROLL):
                    cnt, lastm = scans[u]
                    plsc.addupdate_scatter(hists[p], [digs[p][u]], cnt,
                                           mask=lastm)
            return c

        lax.fori_loop(0, NV // UNROLL, sweep0, 0)

        def permute(p, src, dst):
            hist = hists[p]
            _exclusive_scan(hist, NBINS[p])

            def sweep(i, c):
                raw = [src[pl.ds((i * UNROLL + u) * L, L)]
                       for u in range(UNROLL)]
                ks = [_to_key(v) for v in raw] if p == 0 else raw
                digs = [_digit(k, p) for k in ks]
                scans = [plsc.scan_count(d) for d in digs]
                vals = ks if p < 2 else [_from_key(k) for k in ks]
                for u in range(UNROLL):
                    cnt, lastm = scans[u]
                    d = digs[u]
                    base = plsc.load_gather(hist, [d])
                    nxt = base + cnt
                    plsc.store_scatter(hist, [d], nxt, mask=lastm)
                    plsc.store_scatter(dst, [nxt - 1], vals[u])
                return c

            lax.fori_loop(0, NV // UNROLL, sweep, 0)

        permute(0, src0, pong)
        permute(1, pong, src0)
        permute(2, src0, pong)

    sched_x = [0, 2, 1, 0]
    sched_y = [1, 0, 2, 1]
    in_h = {0: pltpu.async_copy(x_hbm.at[row0], bufs[0], sem_in)}
    out_h = {}
    for r in range(RPW):
        x_buf = bufs[sched_x[r]]
        y_buf = bufs[sched_y[r]]
        in_h[r].wait()
        if r >= 1:
            out_h[r - 1].wait()
        if r + 1 < RPW:
            in_h[r + 1] = pltpu.async_copy(
                x_hbm.at[row0 + (r + 1)], bufs[sched_x[r + 1]], sem_in)
        sort_row(x_buf, y_buf)
        out_h[r] = pltpu.async_copy(y_buf, out_hbm.at[row0 + r], sem_out)
    out_h[RPW - 1].wait()


@jax.jit
def kernel(x):
    xi = lax.bitcast_convert_type(x, jnp.int32)
    mesh = plsc.VectorSubcoreMesh(core_axis_name="c", subcore_axis_name="s")
    sort_rows = pl.kernel(
        _body,
        out_type=jax.ShapeDtypeStruct((ROWS, N), jnp.int32),
        mesh=mesh,
        compiler_params=pltpu.CompilerParams(needs_layout_passes=False),
        scratch_types=[
            pltpu.VMEM((N,), jnp.int32),
            pltpu.VMEM((N,), jnp.int32),
            pltpu.VMEM((N,), jnp.int32),
            pltpu.VMEM((NBINS[0],), jnp.int32),
            pltpu.VMEM((NBINS[1],), jnp.int32),
            pltpu.VMEM((NBINS[2],), jnp.int32),
            pltpu.SemaphoreType.DMA,
            pltpu.SemaphoreType.DMA,
        ],
    )
    oi = sort_rows(xi)
    return lax.bitcast_convert_type(oi, jnp.float32)
